# baseline (speedup 1.0000x reference)
import jax
import jax.numpy as jnp
from jax.experimental import pallas as pl
from jax.experimental.pallas import tpu as pltpu


def _apply_mlp(ps, x):
    for i, p in enumerate(ps):
        x = x @ p["W"] + p["b"]
        if i < len(ps) - 1:
            x = jax.nn.gelu(x, approximate=False)
    return x


def _clusters(coords, batch, nx, ny):
    cx = jnp.clip(jnp.floor(coords[:, 0] * nx).astype(jnp.int32), 0, nx - 1)
    cy = jnp.clip(jnp.floor(coords[:, 1] * ny).astype(jnp.int32), 0, ny - 1)
    return batch.astype(jnp.int32) * (nx * ny) + cx * ny + cy


def _seg_mean(v, seg, num):
    s = jax.ops.segment_sum(v, seg, num_segments=num)
    c = jax.ops.segment_sum(jnp.ones((v.shape[0],), v.dtype), seg, num_segments=num)
    return s / jnp.maximum(c, 1.0)[:, None]


def _dd(p, x, sc, sb, tc, tb, nx, ny, nb):
    ss = _clusters(sc, sb, nx, ny)
    ts = _clusters(tc, tb, nx, ny)
    v = _apply_mlp(p["op_v"], x)
    pooled = _seg_mean(v, ss, nb * nx * ny)
    g = jnp.take(pooled, ts, axis=0)
    return _apply_mlp(p["op_tgt_kernel"], jnp.concatenate([tc, g], axis=1))


def _in(x, batch, nb):
    seg = batch.astype(jnp.int32)
    mean = jnp.take(_seg_mean(x, seg, nb), seg, axis=0)
    var = jnp.take(_seg_mean((x - mean) ** 2, seg, nb), seg, axis=0)
    return (x - mean) / jnp.sqrt(var + 1e-5)


def _block(p, x, sc, sb, tc, tb, nx, ny, nb):
    x = _dd(p["op1"], x, sc, sb, sc, sb, nx, ny, nb)
    x = jax.nn.gelu(_in(x, sb, nb), approximate=False)
    x = _dd(p["op2"], x, sc, sb, tc, tb, nx, ny, nb)
    x = jax.nn.gelu(_in(x, tb, nb), approximate=False)
    return x


def _pool(x, pos, batch, nx, ny, nb):
    sub = _clusters(pos, batch, nx, ny)
    num = nb * nx * ny
    return (_seg_mean(x, sub, num), _seg_mean(pos, sub, num),
            jnp.repeat(jnp.arange(nb, dtype=jnp.int32), nx * ny))


def kernel(x, pos, batch, params):
    B = 4
    x0 = jnp.concatenate([_apply_mlp(params["point_encode"], pos),
                          _apply_mlp(params["lift"], x)], axis=1)
    e1 = _block(params["enc1"], x0, pos, batch, pos, batch, 16, 16, B)
    p1x, p1pos, p1b = _pool(e1, pos, batch, 16, 16, B)
    e2 = _block(params["enc2"], p1x, p1pos, p1b, p1pos, p1b, 8, 8, B)
    p2x, p2pos, p2b = _pool(e2, p1pos, p1b, 8, 8, B)
    e3 = _block(params["enc3"], p2x, p2pos, p2b, p2pos, p2b, 4, 4, B)
    p3x, p3pos, p3b = _pool(e3, p2pos, p2b, 4, 4, B)
    e4 = _block(params["enc4"], p3x, p3pos, p3b, p3pos, p3b, 2, 2, B)
    p4x, p4pos, p4b = _pool(e4, p3pos, p3b, 2, 2, B)
    b = _block(params["bot1"], p4x, p4pos, p4b, p4pos, p4b, 1, 1, B)
    b = _block(params["bot2"], b, p4pos, p4b, p4pos, p4b, 1, 1, B)
    d4 = _block(params["dec4"], jnp.concatenate([b, p4x], axis=1), p4pos, p4b, p3pos, p3b, 2, 2, B)
    d3 = _block(params["dec3"], jnp.concatenate([d4, p3x], axis=1), p3pos, p3b, p2pos, p2b, 4, 4, B)
    d2 = _block(params["dec2"], jnp.concatenate([d3, p2x], axis=1), p2pos, p2b, p1pos, p1b, 8, 8, B)
    d1 = _block(params["dec1"], jnp.concatenate([d2, p1x], axis=1), p1pos, p1b, pos, batch, 16, 16, B)
    d0 = _block(params["dec0"], jnp.concatenate([d1, x0], axis=1), pos, batch, pos, batch, 16, 16, B)
    y = _apply_mlp(params["project"], d0)

    def _ident(y_ref, o_ref):
        o_ref[...] = y_ref[...]

    return pl.pallas_call(
        _ident,
        out_shape=jax.ShapeDtypeStruct(y.shape, y.dtype),
        grid=(50,),
        in_specs=[pl.BlockSpec((2000, 128), lambda i: (i, 0))],
        out_specs=pl.BlockSpec((2000, 128), lambda i: (i, 0)),
    )(y)


# speedup vs baseline: 2.9157x; 2.9157x over previous
import functools

import jax
import jax.numpy as jnp
from jax import lax
from jax.experimental import pallas as pl
from jax.experimental.pallas import tpu as pltpu

_NB = 4
_G = 16
_S = _NB * _G * _G
_R = 2048
_F32 = jnp.float32


def _gelu(x):
    return x * 0.5 * (1.0 + lax.erf(x * 0.7071067811865476))


def _mlpw(p):
    ws = []
    for layer in p:
        ws.append(layer["W"])
        ws.append(layer["b"].reshape(1, -1))
    return ws


def _mlp(x, ws):
    n = len(ws) // 2
    for i in range(n):
        x = jnp.dot(x, ws[2 * i], preferred_element_type=_F32) + ws[2 * i + 1]
        if i < n - 1:
            x = _gelu(x)
    return x


def _dotg0(a, b):
    return lax.dot_general(a, b, (((0,), (0,)), ((), ())),
                           preferred_element_type=_F32)


def _onehot_rows(pos, batch):
    cx = jnp.clip(jnp.floor(pos[:, 0:1] * _G).astype(jnp.int32), 0, _G - 1)
    cy = jnp.clip(jnp.floor(pos[:, 1:2] * _G).astype(jnp.int32), 0, _G - 1)
    sub = batch * (_G * _G) + cx * _G + cy
    ids = lax.broadcasted_iota(jnp.int32, (pos.shape[0], _S), 1)
    return (sub == ids).astype(_F32)


def _bh(batch):
    ids = lax.broadcasted_iota(jnp.int32, (batch.shape[0], 8), 1)
    return (batch == ids).astype(_F32)


def _in_gelu(x, batch, bstats):
    C = x.shape[1]
    s = jnp.dot(_bh(batch), bstats, preferred_element_type=_F32)
    cnt = jnp.maximum(s[:, 2 * C:2 * C + 1], 1.0)
    mean = s[:, :C] / cnt
    var = s[:, C:2 * C] / cnt - mean * mean
    return _gelu((x - mean) / jnp.sqrt(var + 1e-5))


def _acc_init(ref):
    @pl.when(pl.program_id(0) == 0)
    def _():
        ref[...] = jnp.zeros_like(ref)


def _p1_body(x_ref, pos_ref, b_ref, *rest):
    ws = [r[...] for r in rest[:-2]]
    x0_ref, acc_ref = rest[-2], rest[-1]
    pos, b = pos_ref[...], b_ref[...]
    pe = _mlp(pos, ws[0:6])
    lf = _mlp(x_ref[...], ws[6:10])
    x0 = jnp.concatenate([pe, lf], axis=1)
    x0_ref[...] = x0
    v = _mlp(x0, ws[10:14])
    oh = _onehot_rows(pos, b)
    ones = jnp.ones((pos.shape[0], 1), _F32)
    vals = jnp.concatenate([v, pos, ones], axis=1)
    _acc_init(acc_ref)
    acc_ref[...] += _dotg0(oh, vals)


def _pg_body(pos_ref, b_ref, tab_ref, *rest, ccol):
    ws = [r[...] for r in rest[:-2]]
    out_ref, bst_ref = rest[-2], rest[-1]
    pos, b = pos_ref[...], b_ref[...]
    oh = _onehot_rows(pos, b)
    g = jnp.dot(oh, tab_ref[...], preferred_element_type=_F32)
    if ccol is None:
        mean = g
    else:
        mean = g[:, :32] / jnp.maximum(g[:, ccol:ccol + 1], 1.0)
    out = _mlp(jnp.concatenate([pos, mean], axis=1), ws)
    out_ref[...] = out
    ones = jnp.ones((pos.shape[0], 1), _F32)
    sb = jnp.concatenate([out, out * out, ones], axis=1)
    _acc_init(bst_ref)
    bst_ref[...] += _dotg0(_bh(b), sb)


def _ns_body(pos_ref, b_ref, prev_ref, bst_ref, *rest, nws, with_x0):
    k = 1 if with_x0 else 0
    x0 = rest[0][...] if with_x0 else None
    ws = [r[...] for r in rest[k:k + nws]]
    acc_ref = rest[-1]
    pos, b = pos_ref[...], b_ref[...]
    h = _in_gelu(prev_ref[...], b, bst_ref[...])
    if with_x0:
        h = jnp.concatenate([h, x0], axis=1)
    v = _mlp(h, ws) if nws else h
    oh = _onehot_rows(pos, b)
    ones = jnp.ones((pos.shape[0], 1), _F32)
    _acc_init(acc_ref)
    acc_ref[...] += _dotg0(oh, jnp.concatenate([v, ones], axis=1))


def _p11_body(b_ref, prev_ref, bst_ref, *rest):
    ws = [r[...] for r in rest[:-1]]
    y_ref = rest[-1]
    h = _in_gelu(prev_ref[...], b_ref[...], bst_ref[...])
    y_ref[...] = _mlp(h, ws)


def _mid_body(acc1_ref, acc3_ref, *rest):
    out_ref = rest[-1]
    loaded = iter([r[...] for r in rest[:-1]])

    def take4():
        return [next(loaded) for _ in range(4)]

    def ohm(ppos, pb, n):
        m = ppos.shape[0]
        cx = jnp.clip(jnp.floor(ppos[:, 0:1] * n).astype(jnp.int32), 0, n - 1)
        cy = jnp.clip(jnp.floor(ppos[:, 1:2] * n).astype(jnp.int32), 0, n - 1)
        sub = pb * (n * n) + cx * n + cy
        ids = lax.broadcasted_iota(jnp.int32, (m, _NB * n * n), 1)
        return (sub == ids).astype(_F32)

    def bhm(pb):
        ids = lax.broadcasted_iota(jnp.int32, (pb.shape[0], _NB), 1)
        return (pb == ids).astype(_F32)

    def dd(x, s_oh, t_oh, tpos, vws, tws):
        v = _mlp(x, vws)
        c = v.shape[1]
        ones = jnp.ones((x.shape[0], 1), _F32)
        sums = _dotg0(s_oh, jnp.concatenate([v, ones], axis=1))
        g = jnp.dot(t_oh, sums, preferred_element_type=_F32)
        mean = g[:, :c] / jnp.maximum(g[:, c:c + 1], 1.0)
        return _mlp(jnp.concatenate([tpos, mean], axis=1), tws)

    def inorm(x, bho):
        c = x.shape[1]
        ones = jnp.ones((x.shape[0], 1), _F32)
        s = _dotg0(bho, jnp.concatenate([x, x * x, ones], axis=1))
        row = jnp.dot(bho, s, preferred_element_type=_F32)
        cnt = jnp.maximum(row[:, 2 * c:2 * c + 1], 1.0)
        mean = row[:, :c] / cnt
        var = row[:, c:2 * c] / cnt - mean * mean
        return _gelu((x - mean) / jnp.sqrt(var + 1e-5))

    def blockf(x, spos, s_oh, s_bh, tpos, t_oh, t_bh):
        o = dd(x, s_oh, s_oh, spos, take4(), take4())
        o = inorm(o, s_bh)
        o = dd(o, s_oh, t_oh, tpos, take4(), take4())
        return inorm(o, t_bh)

    def pool(x, ppos, oh):
        ones = jnp.ones((x.shape[0], 1), _F32)
        ps = _dotg0(oh, jnp.concatenate([x, ppos, ones], axis=1))
        c = x.shape[1]
        cnt = jnp.maximum(ps[:, c + 2:c + 3], 1.0)
        return ps[:, :c] / cnt, ps[:, c:c + 2] / cnt

    acc1 = acc1_ref[...]
    acc3 = acc3_ref[...]
    cnt1 = jnp.maximum(acc1[:, 34:35], 1.0)
    p1pos = acc1[:, 32:34] / cnt1
    p1x = acc3[:, 0:32] / cnt1
    p1b = lax.broadcasted_iota(jnp.int32, (1024, 1), 0) // 256
    p2b = lax.broadcasted_iota(jnp.int32, (256, 1), 0) // 64
    p3b = lax.broadcasted_iota(jnp.int32, (64, 1), 0) // 16
    p4b = lax.broadcasted_iota(jnp.int32, (16, 1), 0) // 4
    bh1, bh2, bh3, bh4 = bhm(p1b), bhm(p2b), bhm(p3b), bhm(p4b)

    oh_p1_8 = ohm(p1pos, p1b, 8)
    e2 = blockf(p1x, p1pos, oh_p1_8, bh1, p1pos, oh_p1_8, bh1)
    p2x, p2pos = pool(e2, p1pos, oh_p1_8)
    oh_p2_4 = ohm(p2pos, p2b, 4)
    e3 = blockf(p2x, p2pos, oh_p2_4, bh2, p2pos, oh_p2_4, bh2)
    p3x, p3pos = pool(e3, p2pos, oh_p2_4)
    oh_p3_2 = ohm(p3pos, p3b, 2)
    e4 = blockf(p3x, p3pos, oh_p3_2, bh3, p3pos, oh_p3_2, bh3)
    p4x, p4pos = pool(e4, p3pos, oh_p3_2)
    oh_p4_1 = ohm(p4pos, p4b, 1)
    bb = blockf(p4x, p4pos, oh_p4_1, bh4, p4pos, oh_p4_1, bh4)
    bb = blockf(bb, p4pos, oh_p4_1, bh4, p4pos, oh_p4_1, bh4)
    oh_p4_2 = ohm(p4pos, p4b, 2)
    d4 = blockf(jnp.concatenate([bb, p4x], axis=1), p4pos, oh_p4_2,
                bh4, p3pos, oh_p3_2, bh3)
    oh_p3_4 = ohm(p3pos, p3b, 4)
    d3 = blockf(jnp.concatenate([d4, p3x], axis=1), p3pos, oh_p3_4,
                bh3, p2pos, oh_p2_4, bh2)
    oh_p2_8 = ohm(p2pos, p2b, 8)
    d2 = blockf(jnp.concatenate([d3, p2x], axis=1), p2pos, oh_p2_8,
                bh2, p1pos, oh_p1_8, bh1)
    oh_p1_16 = ohm(p1pos, p1b, 16)
    o = dd(jnp.concatenate([d2, p1x], axis=1), oh_p1_16, oh_p1_16,
           p1pos, take4(), take4())
    h = inorm(o, bh1)
    v = _mlp(h, take4())
    ones = jnp.ones((1024, 1), _F32)
    sums = _dotg0(oh_p1_16, jnp.concatenate([v, ones], axis=1))
    out_ref[...] = sums[:, :32] / jnp.maximum(sums[:, 32:33], 1.0)


def _rows(c):
    return pl.BlockSpec((_R, c), lambda i: (i, 0))


def _full(a):
    nd = a.ndim
    return pl.BlockSpec(a.shape, lambda i: (0,) * nd)


def _sds(shape):
    return jax.ShapeDtypeStruct(shape, _F32)


def kernel(x, pos, batch, params):
    n = x.shape[0]
    nt = -(-n // _R)
    npad = nt * _R
    padn = npad - n
    xp = jnp.pad(x, ((0, padn), (0, 0)))
    posp = jnp.pad(pos, ((0, padn), (0, 0)))
    bp = jnp.pad(batch.astype(jnp.int32), (0, padn),
                 constant_values=_NB).reshape(npad, 1)

    def call(body, ins, outs, out_specs):
        specs = []
        for a, kind in ins:
            specs.append(_rows(kind) if isinstance(kind, int) else _full(a))
        return pl.pallas_call(
            body,
            grid=(nt,),
            in_specs=specs,
            out_specs=out_specs,
            out_shape=outs,
        )(*[a for a, _ in ins])

    p = params
    e1o1, e1o2 = p["enc1"]["op1"], p["enc1"]["op2"]
    d0o1, d0o2 = p["dec0"]["op1"], p["dec0"]["op2"]

    ws1 = (_mlpw(p["point_encode"]) + _mlpw(p["lift"]) + _mlpw(e1o1["op_v"]))
    x0, acc1 = call(
        _p1_body,
        [(xp, 128), (posp, 2), (bp, 1)] + [(w, None) for w in ws1],
        [_sds((npad, 96)), _sds((_S, 35))],
        [_rows(96), _full(jnp.zeros((_S, 35)))],
    )

    def gather_pass(tab, tws, ccol):
        return call(
            functools.partial(_pg_body, ccol=ccol),
            [(posp, 2), (bp, 1), (tab, None)] + [(w, None) for w in tws],
            [_sds((npad, 32)), _sds((8, 65))],
            [_rows(32), _full(jnp.zeros((8, 65)))],
        )

    def ns_pass(prev, bst, ws, with_x0=False):
        ins = [(posp, 2), (bp, 1), (prev, 32), (bst, None)]
        if with_x0:
            ins.append((x0, 96))
        ins += [(w, None) for w in ws]
        return call(
            functools.partial(_ns_body, nws=len(ws), with_x0=with_x0),
            ins,
            _sds((_S, 33)),
            _full(jnp.zeros((_S, 33))),
        )

    out1, bst1 = gather_pass(acc1, _mlpw(e1o1["op_tgt_kernel"]), ccol=34)
    acc2 = ns_pass(out1, bst1, _mlpw(e1o2["op_v"]))
    out2, bst2 = gather_pass(acc2, _mlpw(e1o2["op_tgt_kernel"]), ccol=32)
    acc3 = ns_pass(out2, bst2, [])

    mid_ws = []
    for name in ["enc2", "enc3", "enc4", "bot1", "bot2",
                 "dec4", "dec3", "dec2"]:
        for opn in ["op1", "op2"]:
            mid_ws += _mlpw(p[name][opn]["op_v"])
            mid_ws += _mlpw(p[name][opn]["op_tgt_kernel"])
    mid_ws += _mlpw(p["dec1"]["op1"]["op_v"])
    mid_ws += _mlpw(p["dec1"]["op1"]["op_tgt_kernel"])
    mid_ws += _mlpw(p["dec1"]["op2"]["op_v"])
    d1tab = call(
        _mid_body,
        [(acc1, None), (acc3, None)] + [(w, None) for w in mid_ws],
        _sds((_S, 32)),
        _full(jnp.zeros((_S, 32))),
    )

    out_d1, bst3 = gather_pass(d1tab, _mlpw(p["dec1"]["op2"]["op_tgt_kernel"]),
                               ccol=None)
    acc4 = ns_pass(out_d1, bst3, _mlpw(d0o1["op_v"]), with_x0=True)
    out_e, bst4 = gather_pass(acc4, _mlpw(d0o1["op_tgt_kernel"]), ccol=32)
    acc5 = ns_pass(out_e, bst4, _mlpw(d0o2["op_v"]))
    out_f, bst5 = gather_pass(acc5, _mlpw(d0o2["op_tgt_kernel"]), ccol=32)

    y = call(
        _p11_body,
        [(bp, 1), (out_f, 32), (bst5, None)] + [(w, None)
                                                for w in _mlpw(p["project"])],
        _sds((npad, 128)),
        _rows(128),
    )
    return y[:n]
